# baseline (speedup 1.0000x reference)
import functools

import jax
import jax.numpy as jnp
from jax import lax
from jax.experimental import pallas as pl
from jax.experimental.pallas import tpu as pltpu
from jax.experimental.pallas import tpu_sc as plsc

N_NODES = 10000
N_EDGES = 320000
D = 128
OUT = 64
NC = 2
NS = 16
NW = NC * NS
EPW = N_EDGES // NW
LANES = 16
CHUNKS = EPW // LANES
WIN = 10240
WCHUNKS = WIN // LANES
BLK = 16
NBLK = WCHUNKS // BLK
NSLICE = 4
SL_ED = WIN // NSLICE
SL_BLK = NBLK // NSLICE
NOPOS = 1 << 27


def _sc_filter_gather(emb_hbm, edges_hbm, sum_out, deg_out,
                      edgeb, srcwb, idxb, rowsb, accb, degb,
                      degfb, sem_d0, sem_d1, sem_d2, sem_d3, sem_s, sem_g):
    wid = lax.axis_index("s") * NC + lax.axis_index("c")
    base = wid * EPW
    ab = jnp.minimum((base // 128) * 128, N_EDGES - WIN)
    lo = (base - ab) // LANES
    sems = [sem_d0, sem_d1, sem_d2, sem_d3]
    dcs = [
        pltpu.async_copy(edges_hbm.at[1, pl.ds(ab + s * SL_ED, SL_ED)],
                         edgeb.at[pl.ds(s * SL_ED, SL_ED)], sems[s])
        for s in range(NSLICE)
    ]
    scp = pltpu.async_copy(edges_hbm.at[0, pl.ds(ab, WIN)], srcwb, sem_s)

    zf = jnp.zeros((LANES,), jnp.float32)
    for k in range(D // LANES):
        accb[pl.ds(k * LANES, LANES)] = zf
    degb[...] = jnp.zeros((LANES,), jnp.int32)

    ones = jnp.ones((LANES,), jnp.int32)

    def neutralize(c, carry):
        edgeb[pl.ds(c * LANES, LANES)] = ones
        return carry

    def acc_row(r, c2):
        for k in range(D // LANES):
            sl = pl.ds(k * LANES, LANES)
            accb[sl] = accb[sl] + rowsb[r, sl]
        return c2

    NACC = 8

    def screen_blk(b, carry):
        st = list(carry)
        off0 = b * BLK * LANES
        for t in range(BLK):
            c = b * BLK + t
            dv = edgeb[pl.ds(off0 + t * LANES, LANES)]
            m = dv == 0
            a = t % NACC
            st[2 * a] = st[2 * a] + jnp.where(m, 1, 0).astype(jnp.int32)
            st[2 * a + 1] = jnp.minimum(st[2 * a + 1],
                                        jnp.where(m, c, NOPOS))
        return tuple(st)

    cv0 = jnp.zeros((LANES,), jnp.int32)
    pm0 = jnp.full((LANES,), NOPOS, jnp.int32)
    st = (cv0, pm0) * NACC
    for s in range(NSLICE):
        dcs[s].wait()
        if s == 0:
            lax.fori_loop(0, lo, neutralize, 0)
        if s == NSLICE - 1:
            lax.fori_loop(lo + CHUNKS, WCHUNKS, neutralize, 0)
        st = lax.fori_loop(s * SL_BLK, (s + 1) * SL_BLK, screen_blk, st)
    cvs = list(st[0::2])
    pms = list(st[1::2])
    while len(cvs) > 1:
        cvs = [cvs[i] + cvs[i + 1] for i in range(0, len(cvs), 2)]
        pms = [jnp.minimum(pms[i], pms[i + 1]) for i in range(0, len(pms), 2)]
    cv = cvs[0]
    pmin = pms[0]

    def fine(c, carry):
        off = c * LANES
        dv = edgeb[pl.ds(off, LANES)]
        m = dv == 0
        mi = jnp.where(m, 1, 0).astype(jnp.int32)
        cnt = jnp.sum(mi)

        @pl.when(cnt > 0)
        def _():
            sv = srcwb[pl.ds(off, LANES)]
            idxb[...] = jnp.zeros((LANES,), jnp.int32)
            plsc.store_compressed(idxb.at[pl.ds(0, LANES)], sv, mask=m)
            degb[...] = degb[...] + mi
            pltpu.async_copy(emb_hbm.at[idxb], rowsb, sem_g).wait()
            lax.fori_loop(0, cnt, acc_row, 0)

        return carry

    def rescan(b, carry):
        off0 = b * BLK * LANES
        mn = edgeb[pl.ds(off0, LANES)]
        for t in range(1, BLK):
            mn = jnp.minimum(mn, edgeb[pl.ds(off0 + t * LANES, LANES)])

        @pl.when(jnp.min(mn) == 0)
        def _():
            lax.fori_loop(b * BLK, (b + 1) * BLK, fine, 0)

        return carry

    total = jnp.sum(cv)
    scp.wait()

    @pl.when(total > 0)
    def _():
        mx = jnp.max(cv)

        @pl.when(mx == 1)
        def _():
            iot = lax.iota(jnp.int32, 16)
            hasm = cv == 1
            pm = jnp.where(hasm, pmin, 0)
            wpos = pm * LANES + iot
            srcv = plsc.load_gather(srcwb, [wpos])
            srcm = jnp.where(hasm, srcv, 0)
            idxb[...] = jnp.zeros((LANES,), jnp.int32)
            plsc.store_compressed(idxb.at[pl.ds(0, LANES)], srcm, mask=hasm)
            degb[...] = cv
            pltpu.async_copy(emb_hbm.at[idxb], rowsb, sem_g).wait()
            lax.fori_loop(0, total, acc_row, 0)

        @pl.when(mx > 1)
        def _():
            lax.fori_loop(0, NBLK, rescan, 0)

    pltpu.sync_copy(accb, sum_out.at[wid])
    dt = jnp.sum(degb[...]).astype(jnp.float32)
    for k in range(D // LANES):
        degfb[pl.ds(k * LANES, LANES)] = jnp.full((LANES,), dt, jnp.float32)
    pltpu.sync_copy(degfb, deg_out.at[wid])


def _tc_finish(part_ref, deg_ref, emb_ref, ws_ref, wn_ref, bs_ref,
               wc_ref, bc_ref, out_ref):
    s = jnp.sum(part_ref[...], axis=0, keepdims=True)
    deg = jnp.sum(deg_ref[...], axis=0, keepdims=True)[0:1, 0:1]
    agg = s / jnp.maximum(deg, 1.0)
    e0 = emb_ref[0:1, :]
    h = jnp.maximum(
        jnp.dot(e0, ws_ref[...], preferred_element_type=jnp.float32)
        + jnp.dot(agg, wn_ref[...], preferred_element_type=jnp.float32)
        + bs_ref[...][None, :], 0.0)
    out_ref[...] = (jnp.dot(h, wc_ref[...], preferred_element_type=jnp.float32)
                    + bc_ref[...][None, :])


def kernel(embedding, edges, W_self, W_neigh, b_sage, W_cls, b_cls):
    edges = edges.astype(jnp.int32)

    mesh = plsc.VectorSubcoreMesh(core_axis_name="c", subcore_axis_name="s")
    sc_call = functools.partial(
        pl.kernel,
        mesh=mesh,
        compiler_params=pltpu.CompilerParams(needs_layout_passes=False),
        out_type=(
            jax.ShapeDtypeStruct((NW, D), jnp.float32),
            jax.ShapeDtypeStruct((NW, D), jnp.float32),
        ),
        scratch_types=[
            pltpu.VMEM((WIN,), jnp.int32),
            pltpu.VMEM((WIN,), jnp.int32),
            pltpu.VMEM((LANES,), jnp.int32),
            pltpu.VMEM((LANES, D), jnp.float32),
            pltpu.VMEM((D,), jnp.float32),
            pltpu.VMEM((LANES,), jnp.int32),
            pltpu.VMEM((D,), jnp.float32),
            pltpu.SemaphoreType.DMA,
            pltpu.SemaphoreType.DMA,
            pltpu.SemaphoreType.DMA,
            pltpu.SemaphoreType.DMA,
            pltpu.SemaphoreType.DMA,
            pltpu.SemaphoreType.DMA,
        ],
    )
    partials, degs = sc_call(_sc_filter_gather)(embedding, edges)

    out = pl.pallas_call(
        _tc_finish,
        out_shape=jax.ShapeDtypeStruct((1, OUT), jnp.float32),
        grid=(1,),
        in_specs=[
            pl.BlockSpec((NW, D), lambda i: (0, 0)),
            pl.BlockSpec((NW, D), lambda i: (0, 0)),
            pl.BlockSpec((8, D), lambda i: (0, 0)),
            pl.BlockSpec((D, D), lambda i: (0, 0)),
            pl.BlockSpec((D, D), lambda i: (0, 0)),
            pl.BlockSpec((D,), lambda i: (0,)),
            pl.BlockSpec((D, OUT), lambda i: (0, 0)),
            pl.BlockSpec((OUT,), lambda i: (0,)),
        ],
        out_specs=pl.BlockSpec((1, OUT), lambda i: (0, 0)),
    )(partials, degs, embedding, W_self, W_neigh, b_sage, W_cls, b_cls)

    return out


# speedup vs baseline: 1.0999x; 1.0999x over previous
import functools

import jax
import jax.numpy as jnp
from jax import lax
from jax.experimental import pallas as pl
from jax.experimental.pallas import tpu as pltpu
from jax.experimental.pallas import tpu_sc as plsc

N_NODES = 10000
N_EDGES = 320000
D = 128
OUT = 64
NC = 2
NS = 16
NW = NC * NS
EPW = N_EDGES // NW
LANES = 16
CHUNKS = EPW // LANES
WIN = 10240
WCHUNKS = WIN // LANES
BLK = 16
NBLK = WCHUNKS // BLK
NSLICE = 4
SL_ED = WIN // NSLICE
SL_BLK = NBLK // NSLICE
NOPOS = 1 << 27


def _sc_filter_gather(emb_hbm, edges_hbm, sum_out, deg_out,
                      edgeb, srcwb, idxb, rowsb, accb, degb,
                      degfb, sem_d0, sem_d1, sem_d2, sem_d3, sem_s, sem_g):
    wid = lax.axis_index("s") * NC + lax.axis_index("c")
    base = wid * EPW
    ab = jnp.minimum((base // 128) * 128, N_EDGES - WIN)
    lo = (base - ab) // LANES
    sems = [sem_d0, sem_d1, sem_d2, sem_d3]
    dcs = [
        pltpu.async_copy(edges_hbm.at[1, pl.ds(ab + s * SL_ED, SL_ED)],
                         edgeb.at[pl.ds(s * SL_ED, SL_ED)], sems[s])
        for s in range(NSLICE)
    ]
    scp = pltpu.async_copy(edges_hbm.at[0, pl.ds(ab, WIN)], srcwb, sem_s)

    zf = jnp.zeros((LANES,), jnp.float32)
    for k in range(D // LANES):
        accb[pl.ds(k * LANES, LANES)] = zf
    degb[...] = jnp.zeros((LANES,), jnp.int32)

    ones = jnp.ones((LANES,), jnp.int32)

    def neutralize(c, carry):
        edgeb[pl.ds(c * LANES, LANES)] = ones
        return carry

    def acc_row(r, c2):
        for k in range(D // LANES):
            sl = pl.ds(k * LANES, LANES)
            accb[sl] = accb[sl] + rowsb[r, sl]
        return c2

    def screen_blk(b, carry):
        cnt, bpmin, bpmax = carry
        off0 = b * BLK * LANES
        mns = [edgeb[pl.ds(off0 + t * LANES, LANES)] for t in range(BLK)]
        while len(mns) > 1:
            mns = [jnp.minimum(mns[i], mns[i + 1])
                   for i in range(0, len(mns), 2)]
        m = mns[0] == 0
        cnt = cnt + jnp.where(m, 1, 0).astype(jnp.int32)
        bpmin = jnp.minimum(bpmin, jnp.where(m, b, NOPOS))
        bpmax = jnp.maximum(bpmax, jnp.where(m, b, -1))
        return cnt, bpmin, bpmax

    st = (jnp.zeros((LANES,), jnp.int32),
          jnp.full((LANES,), NOPOS, jnp.int32),
          jnp.full((LANES,), -1, jnp.int32))
    for s in range(NSLICE):
        dcs[s].wait()
        if s == 0:
            lax.fori_loop(0, lo, neutralize, 0)
        if s == NSLICE - 1:
            lax.fori_loop(lo + CHUNKS, WCHUNKS, neutralize, 0)
        st = lax.fori_loop(s * SL_BLK, (s + 1) * SL_BLK, screen_blk, st)
    cnt, bpmin, bpmax = st

    def fine(c, carry):
        off = c * LANES
        dv = edgeb[pl.ds(off, LANES)]
        m = dv == 0
        mi = jnp.where(m, 1, 0).astype(jnp.int32)
        cnt = jnp.sum(mi)

        @pl.when(cnt > 0)
        def _():
            sv = srcwb[pl.ds(off, LANES)]
            idxb[...] = jnp.zeros((LANES,), jnp.int32)
            plsc.store_compressed(idxb.at[pl.ds(0, LANES)], sv, mask=m)
            degb[...] = degb[...] + mi
            pltpu.async_copy(emb_hbm.at[idxb], rowsb, sem_g).wait()
            lax.fori_loop(0, cnt, acc_row, 0)

        return carry

    def rescan(b, carry):
        off0 = b * BLK * LANES
        mn = edgeb[pl.ds(off0, LANES)]
        for t in range(1, BLK):
            mn = jnp.minimum(mn, edgeb[pl.ds(off0 + t * LANES, LANES)])

        @pl.when(jnp.min(mn) == 0)
        def _():
            lax.fori_loop(b * BLK, (b + 1) * BLK, fine, 0)

        return carry

    total_blk = jnp.sum(cnt)
    scp.wait()

    @pl.when(total_blk > 0)
    def _():
        mxb = jnp.max(cnt)

        @pl.when(mxb <= 2)
        def _():
            iot = lax.iota(jnp.int32, 16)
            onlyb = cnt == 2
            blkA = jnp.where(cnt >= 1, bpmin, 0)
            blkB = jnp.where(onlyb, bpmax, 0)

            def resolve(blkv, emask):
                cva = jnp.zeros((LANES,), jnp.int32)
                pma = jnp.full((LANES,), NOPOS, jnp.int32)
                for t in range(BLK):
                    cvec = blkv * BLK + t
                    dv = plsc.load_gather(edgeb, [cvec * LANES + iot])
                    m = dv == 0
                    if emask is not None:
                        m = jnp.logical_and(m, emask)
                    cva = cva + jnp.where(m, 1, 0).astype(jnp.int32)
                    pma = jnp.minimum(pma, jnp.where(m, cvec, NOPOS))
                return cva, pma

            cvA, pmA = resolve(blkA, None)
            cvB, pmB = resolve(blkB, onlyb)
            mxc = jnp.maximum(jnp.max(cvA), jnp.max(cvB))

            @pl.when(mxc <= 1)
            def _():
                degb[...] = cvA + cvB
                hasA = cvA == 1
                pa = jnp.where(hasA, pmA, 0)
                sva = plsc.load_gather(srcwb, [pa * LANES + iot])
                sva = jnp.where(hasA, sva, 0)
                idxb[...] = jnp.zeros((LANES,), jnp.int32)
                plsc.store_compressed(idxb.at[pl.ds(0, LANES)], sva,
                                      mask=hasA)
                ta = jnp.sum(cvA)
                pltpu.async_copy(emb_hbm.at[idxb], rowsb, sem_g).wait()
                lax.fori_loop(0, ta, acc_row, 0)

                tb = jnp.sum(cvB)

                @pl.when(tb > 0)
                def _():
                    hasB = cvB == 1
                    pb = jnp.where(hasB, pmB, 0)
                    svb = plsc.load_gather(srcwb, [pb * LANES + iot])
                    svb = jnp.where(hasB, svb, 0)
                    idxb[...] = jnp.zeros((LANES,), jnp.int32)
                    plsc.store_compressed(idxb.at[pl.ds(0, LANES)], svb,
                                          mask=hasB)
                    pltpu.async_copy(emb_hbm.at[idxb], rowsb, sem_g).wait()
                    lax.fori_loop(0, tb, acc_row, 0)

            @pl.when(mxc > 1)
            def _():
                lax.fori_loop(0, NBLK, rescan, 0)

        @pl.when(mxb > 2)
        def _():
            lax.fori_loop(0, NBLK, rescan, 0)

    pltpu.sync_copy(accb, sum_out.at[wid])
    dt = jnp.sum(degb[...]).astype(jnp.float32)
    for k in range(D // LANES):
        degfb[pl.ds(k * LANES, LANES)] = jnp.full((LANES,), dt, jnp.float32)
    pltpu.sync_copy(degfb, deg_out.at[wid])


def _tc_finish(part_ref, deg_ref, emb_ref, ws_ref, wn_ref, bs_ref,
               wc_ref, bc_ref, out_ref):
    s = jnp.sum(part_ref[...], axis=0, keepdims=True)
    deg = jnp.sum(deg_ref[...], axis=0, keepdims=True)[0:1, 0:1]
    agg = s / jnp.maximum(deg, 1.0)
    e0 = emb_ref[0:1, :]
    h = jnp.maximum(
        jnp.dot(e0, ws_ref[...], preferred_element_type=jnp.float32)
        + jnp.dot(agg, wn_ref[...], preferred_element_type=jnp.float32)
        + bs_ref[...][None, :], 0.0)
    out_ref[...] = (jnp.dot(h, wc_ref[...], preferred_element_type=jnp.float32)
                    + bc_ref[...][None, :])


def kernel(embedding, edges, W_self, W_neigh, b_sage, W_cls, b_cls):
    edges = edges.astype(jnp.int32)

    mesh = plsc.VectorSubcoreMesh(core_axis_name="c", subcore_axis_name="s")
    sc_call = functools.partial(
        pl.kernel,
        mesh=mesh,
        compiler_params=pltpu.CompilerParams(needs_layout_passes=False),
        out_type=(
            jax.ShapeDtypeStruct((NW, D), jnp.float32),
            jax.ShapeDtypeStruct((NW, D), jnp.float32),
        ),
        scratch_types=[
            pltpu.VMEM((WIN,), jnp.int32),
            pltpu.VMEM((WIN,), jnp.int32),
            pltpu.VMEM((LANES,), jnp.int32),
            pltpu.VMEM((LANES, D), jnp.float32),
            pltpu.VMEM((D,), jnp.float32),
            pltpu.VMEM((LANES,), jnp.int32),
            pltpu.VMEM((D,), jnp.float32),
            pltpu.SemaphoreType.DMA,
            pltpu.SemaphoreType.DMA,
            pltpu.SemaphoreType.DMA,
            pltpu.SemaphoreType.DMA,
            pltpu.SemaphoreType.DMA,
            pltpu.SemaphoreType.DMA,
        ],
    )
    partials, degs = sc_call(_sc_filter_gather)(embedding, edges)

    out = pl.pallas_call(
        _tc_finish,
        out_shape=jax.ShapeDtypeStruct((1, OUT), jnp.float32),
        grid=(1,),
        in_specs=[
            pl.BlockSpec((NW, D), lambda i: (0, 0)),
            pl.BlockSpec((NW, D), lambda i: (0, 0)),
            pl.BlockSpec((8, D), lambda i: (0, 0)),
            pl.BlockSpec((D, D), lambda i: (0, 0)),
            pl.BlockSpec((D, D), lambda i: (0, 0)),
            pl.BlockSpec((D,), lambda i: (0,)),
            pl.BlockSpec((D, OUT), lambda i: (0, 0)),
            pl.BlockSpec((OUT,), lambda i: (0,)),
        ],
        out_specs=pl.BlockSpec((1, OUT), lambda i: (0, 0)),
    )(partials, degs, embedding, W_self, W_neigh, b_sage, W_cls, b_cls)

    return out
